# Initial kernel scaffold; baseline (speedup 1.0000x reference)
import jax
import jax.numpy as jnp
from jax.experimental import pallas as pl


def kernel(node_feats, edge_feats, node_hidden, edge_index, W_node, W_edge, W_ih, W_hh, b_ih, b_hh):
    raise NotImplementedError("write your pallas kernel here")


# speedup vs baseline: 3.0944x; 3.0944x over previous
import functools
import jax
import jax.numpy as jnp
from jax import lax
from jax.experimental import pallas as pl
from jax.experimental.pallas import tpu as pltpu
from jax.experimental.pallas import tpu_sc as plsc

N, E, D, DE, H = 10000, 320000, 128, 16, 128
NC, NS, L = 2, 16, 16
NW = NC * NS
CH = 128
NCHUNKS = E // CH
ROWS_PER_SUB = 624


def _node_prep_body(nf_ref, nh_ref, wn_ref, p_ref, r_ref):
    np_blk = lax.dot_general(nf_ref[...], wn_ref[...],
                             (((1,), (1,)), ((), ())),
                             preferred_element_type=jnp.float32)
    p = jnp.exp(np_blk - jnp.max(np_blk, axis=1, keepdims=True))
    p_ref[...] = p
    r_ref[...] = nh_ref[...] * p


def _edge_prep_body(ef_ref, we_ref, q_ref):
    ep = lax.dot_general(ef_ref[...], we_ref[...],
                         (((1,), (1,)), ((), ())),
                         preferred_element_type=jnp.float32)
    q_ref[...] = jnp.exp(ep - jnp.max(ep, axis=1, keepdims=True))


def _gru_body(hp_ref, nh_ref, wih_ref, whh_ref, bih_ref, bhh_ref, out_ref):
    h_new = hp_ref[0] + hp_ref[1]
    h = nh_ref[...]
    gi = lax.dot_general(h_new, wih_ref[...], (((1,), (1,)), ((), ())),
                         preferred_element_type=jnp.float32) + bih_ref[...]
    gh = lax.dot_general(h, whh_ref[...], (((1,), (1,)), ((), ())),
                         preferred_element_type=jnp.float32) + bhh_ref[...]
    r = jax.nn.sigmoid(gi[:, :H] + gh[:, :H])
    z = jax.nn.sigmoid(gi[:, H:2 * H] + gh[:, H:2 * H])
    n = jnp.tanh(gi[:, 2 * H:] + r * gh[:, 2 * H:])
    out_ref[...] = (1.0 - z) * n + z * h


def _sc_body(p_hbm, r_hbm, q_hbm, src_hbm, dst_hbm, zero_hbm, out_hbm,
             src_v, dst_v, p_v, r_v, q_v, acc_sh, sem0, sem1, sem2):
    cid = lax.axis_index("c")
    sid = lax.axis_index("s")
    wid = sid * NC + cid

    @pl.when(sid == 0)
    def _():
        pltpu.sync_copy(zero_hbm, acc_sh)

    plsc.subcore_barrier()

    base_chunks = NCHUNKS // NW
    extra = NCHUNKS % NW
    n_chunks = base_chunks + jnp.where(wid < extra, 1, 0)

    def chunk_body(k, carry):
        base = (wid + k * NW) * CH
        pltpu.sync_copy(src_hbm.at[pl.ds(base, CH)], src_v)
        pltpu.sync_copy(dst_hbm.at[pl.ds(base, CH)], dst_v)
        cp_p = pltpu.async_copy(p_hbm.at[src_v], p_v, sem0)
        cp_r = pltpu.async_copy(r_hbm.at[src_v], r_v, sem1)
        cp_q = pltpu.async_copy(q_hbm.at[pl.ds(base, CH)], q_v, sem2)
        cp_p.wait()
        cp_q.wait()

        lanes = lax.iota(jnp.int32, L)

        def edge_body(i, c):
            acc = p_v[i, pl.ds(0, L)] * q_v[i, pl.ds(0, L)]
            for j in range(1, H // L):
                acc = acc + p_v[i, pl.ds(j * L, L)] * q_v[i, pl.ds(j * L, L)]
            dnums = lax.GatherDimensionNumbers(
                offset_dims=(), collapsed_slice_dims=(0,),
                start_index_map=(0,))
            for step in (1, 2, 4, 8):
                perm = lax.gather(
                    acc, (lanes ^ step).reshape(L, 1), dnums,
                    slice_sizes=(1,),
                    mode=lax.GatherScatterMode.PROMISE_IN_BOUNDS)
                acc = acc + perm
            s = 1.0 / acc
            for j in range(H // L):
                q_v[i, pl.ds(j * L, L)] = (
                    r_v[i, pl.ds(j * L, L)] * q_v[i, pl.ds(j * L, L)] * s)
            return c

        cp_r.wait()
        lax.fori_loop(0, CH, edge_body, 0)
        pltpu.sync_copy(q_v, acc_sh.at[dst_v], add=True)
        return carry

    lax.fori_loop(0, n_chunks, chunk_body, 0)

    plsc.subcore_barrier()
    row0 = sid * ROWS_PER_SUB
    pltpu.sync_copy(acc_sh.at[pl.ds(row0, ROWS_PER_SUB)],
                    out_hbm.at[cid, pl.ds(row0, ROWS_PER_SUB)])

    tail0 = NS * ROWS_PER_SUB
    tail = N - tail0

    @pl.when(sid == 0)
    def _():
        pltpu.sync_copy(acc_sh.at[pl.ds(tail0, tail)],
                        out_hbm.at[cid, pl.ds(tail0, tail)])


_sc_scatter = functools.partial(
    pl.kernel,
    mesh=plsc.VectorSubcoreMesh(core_axis_name="c", subcore_axis_name="s"),
    out_type=jax.ShapeDtypeStruct((NC, N, H), jnp.float32),
    scratch_types=[
        pltpu.VMEM((CH,), jnp.int32),
        pltpu.VMEM((CH,), jnp.int32),
        pltpu.VMEM((CH, H), jnp.float32),
        pltpu.VMEM((CH, H), jnp.float32),
        pltpu.VMEM((CH, H), jnp.float32),
        pltpu.VMEM_SHARED((N, H), jnp.float32),
        pltpu.SemaphoreType.DMA,
        pltpu.SemaphoreType.DMA,
        pltpu.SemaphoreType.DMA,
    ],
)(_sc_body)


def kernel(node_feats, edge_feats, node_hidden, edge_index, W_node, W_edge,
           W_ih, W_hh, b_ih, b_hh):
    src = edge_index[0]
    dst = edge_index[1]

    BN = 1000
    p_arr, r_arr = pl.pallas_call(
        _node_prep_body,
        grid=(N // BN,),
        in_specs=[pl.BlockSpec((BN, D), lambda i: (i, 0)),
                  pl.BlockSpec((BN, H), lambda i: (i, 0)),
                  pl.BlockSpec((H, D), lambda i: (0, 0))],
        out_specs=[pl.BlockSpec((BN, H), lambda i: (i, 0)),
                   pl.BlockSpec((BN, H), lambda i: (i, 0))],
        out_shape=[jax.ShapeDtypeStruct((N, H), jnp.float32),
                   jax.ShapeDtypeStruct((N, H), jnp.float32)],
    )(node_feats, node_hidden, W_node)

    BE = 4000
    q_arr = pl.pallas_call(
        _edge_prep_body,
        grid=(E // BE,),
        in_specs=[pl.BlockSpec((BE, DE), lambda i: (i, 0)),
                  pl.BlockSpec((H, DE), lambda i: (0, 0))],
        out_specs=pl.BlockSpec((BE, H), lambda i: (i, 0)),
        out_shape=jax.ShapeDtypeStruct((E, H), jnp.float32),
    )(edge_feats, W_edge)

    zeros = jnp.zeros((N, H), jnp.float32)
    hp = _sc_scatter(p_arr, r_arr, q_arr, src, dst, zeros)

    BG = 1000
    out = pl.pallas_call(
        _gru_body,
        grid=(N // BG,),
        in_specs=[pl.BlockSpec((2, BG, H), lambda i: (0, i, 0)),
                  pl.BlockSpec((BG, H), lambda i: (i, 0)),
                  pl.BlockSpec((3 * H, H), lambda i: (0, 0)),
                  pl.BlockSpec((3 * H, H), lambda i: (0, 0)),
                  pl.BlockSpec((1, 3 * H), lambda i: (0, 0)),
                  pl.BlockSpec((1, 3 * H), lambda i: (0, 0))],
        out_specs=pl.BlockSpec((BG, H), lambda i: (i, 0)),
        out_shape=jax.ShapeDtypeStruct((N, H), jnp.float32),
    )(hp, node_hidden, W_ih, W_hh, b_ih.reshape(1, 3 * H),
      b_hh.reshape(1, 3 * H))
    return out


# speedup vs baseline: 3.2548x; 1.0518x over previous
import functools
import jax
import jax.numpy as jnp
from jax import lax
from jax.experimental import pallas as pl
from jax.experimental.pallas import tpu as pltpu
from jax.experimental.pallas import tpu_sc as plsc

N, E, D, DE, H = 10000, 320000, 128, 16, 128
NC, NS, L = 2, 16, 16
NW = NC * NS
CH = 128
NCHUNKS = E // CH
ROWS_PER_SUB = 624


def _node_prep_body(nf_ref, nh_ref, wn_ref, p_ref, r_ref):
    np_blk = lax.dot_general(nf_ref[...], wn_ref[...],
                             (((1,), (1,)), ((), ())),
                             preferred_element_type=jnp.float32)
    p = jnp.exp(np_blk - jnp.max(np_blk, axis=1, keepdims=True))
    p_ref[...] = p
    r_ref[...] = nh_ref[...] * p


def _edge_prep_body(ef_ref, we_ref, q_ref):
    ep = lax.dot_general(ef_ref[...], we_ref[...],
                         (((1,), (1,)), ((), ())),
                         preferred_element_type=jnp.float32)
    q_ref[...] = jnp.exp(ep - jnp.max(ep, axis=1, keepdims=True))


def _gru_body(hp_ref, nh_ref, wih_ref, whh_ref, bih_ref, bhh_ref, out_ref):
    h_new = hp_ref[0] + hp_ref[1]
    h = nh_ref[...]
    gi = lax.dot_general(h_new, wih_ref[...], (((1,), (1,)), ((), ())),
                         preferred_element_type=jnp.float32) + bih_ref[...]
    gh = lax.dot_general(h, whh_ref[...], (((1,), (1,)), ((), ())),
                         preferred_element_type=jnp.float32) + bhh_ref[...]
    r = jax.nn.sigmoid(gi[:, :H] + gh[:, :H])
    z = jax.nn.sigmoid(gi[:, H:2 * H] + gh[:, H:2 * H])
    n = jnp.tanh(gi[:, 2 * H:] + r * gh[:, 2 * H:])
    out_ref[...] = (1.0 - z) * n + z * h


def _sc_body(p_hbm, r_hbm, q_hbm, src_hbm, dst_hbm, zero_hbm, out_hbm,
             src_v, dst_v, p_v, r_v, q_v, acc_sh, sem0, sem1, sem2):
    cid = lax.axis_index("c")
    sid = lax.axis_index("s")
    wid = sid * NC + cid

    @pl.when(sid == 0)
    def _():
        pltpu.sync_copy(zero_hbm, acc_sh)

    plsc.subcore_barrier()

    base_chunks = NCHUNKS // NW
    extra = NCHUNKS % NW
    n_chunks = base_chunks + jnp.where(wid < extra, 1, 0)

    def chunk_body(k, carry):
        base = (wid + k * NW) * CH
        pltpu.sync_copy(src_hbm.at[pl.ds(base, CH)], src_v)
        pltpu.sync_copy(dst_hbm.at[pl.ds(base, CH)], dst_v)
        cp_p = pltpu.async_copy(p_hbm.at[src_v], p_v, sem0)
        cp_r = pltpu.async_copy(r_hbm.at[src_v], r_v, sem1)
        cp_q = pltpu.async_copy(q_hbm.at[pl.ds(base, CH)], q_v, sem2)
        cp_p.wait()
        cp_q.wait()
        cp_r.wait()

        lanes = lax.iota(jnp.int32, L)

        @plsc.parallel_loop(0, CH, unroll=2)
        def edge_body(i):
            acc = p_v[i, pl.ds(0, L)] * q_v[i, pl.ds(0, L)]
            for j in range(1, H // L):
                acc = acc + p_v[i, pl.ds(j * L, L)] * q_v[i, pl.ds(j * L, L)]
            dnums = lax.GatherDimensionNumbers(
                offset_dims=(), collapsed_slice_dims=(0,),
                start_index_map=(0,))
            for step in (1, 2, 4, 8):
                perm = lax.gather(
                    acc, (lanes ^ step).reshape(L, 1), dnums,
                    slice_sizes=(1,),
                    mode=lax.GatherScatterMode.PROMISE_IN_BOUNDS)
                acc = acc + perm
            s = 1.0 / acc
            for j in range(H // L):
                q_v[i, pl.ds(j * L, L)] = (
                    r_v[i, pl.ds(j * L, L)] * q_v[i, pl.ds(j * L, L)] * s)

        pltpu.sync_copy(q_v, acc_sh.at[dst_v], add=True)
        return carry

    lax.fori_loop(0, n_chunks, chunk_body, 0)

    plsc.subcore_barrier()
    row0 = sid * ROWS_PER_SUB
    pltpu.sync_copy(acc_sh.at[pl.ds(row0, ROWS_PER_SUB)],
                    out_hbm.at[cid, pl.ds(row0, ROWS_PER_SUB)])

    tail0 = NS * ROWS_PER_SUB
    tail = N - tail0

    @pl.when(sid == 0)
    def _():
        pltpu.sync_copy(acc_sh.at[pl.ds(tail0, tail)],
                        out_hbm.at[cid, pl.ds(tail0, tail)])


_sc_scatter = functools.partial(
    pl.kernel,
    mesh=plsc.VectorSubcoreMesh(core_axis_name="c", subcore_axis_name="s"),
    out_type=jax.ShapeDtypeStruct((NC, N, H), jnp.float32),
    scratch_types=[
        pltpu.VMEM((CH,), jnp.int32),
        pltpu.VMEM((CH,), jnp.int32),
        pltpu.VMEM((CH, H), jnp.float32),
        pltpu.VMEM((CH, H), jnp.float32),
        pltpu.VMEM((CH, H), jnp.float32),
        pltpu.VMEM_SHARED((N, H), jnp.float32),
        pltpu.SemaphoreType.DMA,
        pltpu.SemaphoreType.DMA,
        pltpu.SemaphoreType.DMA,
    ],
)(_sc_body)


def kernel(node_feats, edge_feats, node_hidden, edge_index, W_node, W_edge,
           W_ih, W_hh, b_ih, b_hh):
    src = edge_index[0]
    dst = edge_index[1]

    BN = 1000
    p_arr, r_arr = pl.pallas_call(
        _node_prep_body,
        grid=(N // BN,),
        in_specs=[pl.BlockSpec((BN, D), lambda i: (i, 0)),
                  pl.BlockSpec((BN, H), lambda i: (i, 0)),
                  pl.BlockSpec((H, D), lambda i: (0, 0))],
        out_specs=[pl.BlockSpec((BN, H), lambda i: (i, 0)),
                   pl.BlockSpec((BN, H), lambda i: (i, 0))],
        out_shape=[jax.ShapeDtypeStruct((N, H), jnp.float32),
                   jax.ShapeDtypeStruct((N, H), jnp.float32)],
    )(node_feats, node_hidden, W_node)

    BE = 4000
    q_arr = pl.pallas_call(
        _edge_prep_body,
        grid=(E // BE,),
        in_specs=[pl.BlockSpec((BE, DE), lambda i: (i, 0)),
                  pl.BlockSpec((H, DE), lambda i: (0, 0))],
        out_specs=pl.BlockSpec((BE, H), lambda i: (i, 0)),
        out_shape=jax.ShapeDtypeStruct((E, H), jnp.float32),
    )(edge_feats, W_edge)

    zeros = jnp.zeros((N, H), jnp.float32)
    hp = _sc_scatter(p_arr, r_arr, q_arr, src, dst, zeros)

    BG = 1000
    out = pl.pallas_call(
        _gru_body,
        grid=(N // BG,),
        in_specs=[pl.BlockSpec((2, BG, H), lambda i: (0, i, 0)),
                  pl.BlockSpec((BG, H), lambda i: (i, 0)),
                  pl.BlockSpec((3 * H, H), lambda i: (0, 0)),
                  pl.BlockSpec((3 * H, H), lambda i: (0, 0)),
                  pl.BlockSpec((1, 3 * H), lambda i: (0, 0)),
                  pl.BlockSpec((1, 3 * H), lambda i: (0, 0))],
        out_specs=pl.BlockSpec((BG, H), lambda i: (i, 0)),
        out_shape=jax.ShapeDtypeStruct((N, H), jnp.float32),
    )(hp, node_hidden, W_ih, W_hh, b_ih.reshape(1, 3 * H),
      b_hh.reshape(1, 3 * H))
    return out


# speedup vs baseline: 4.5531x; 1.3989x over previous
import functools
import jax
import jax.numpy as jnp
from jax import lax
from jax.experimental import pallas as pl
from jax.experimental.pallas import tpu as pltpu
from jax.experimental.pallas import tpu_sc as plsc

N, E, D, DE, H = 10000, 320000, 128, 16, 128
NC, NS, L = 2, 16, 16
NW = NC * NS
CH = 64
NCHUNKS = E // CH
BASE_CHUNKS = NCHUNKS // NW
EXTRA = NCHUNKS % NW
ROWS_PER_SUB = 624


def _node_prep_body(nf_ref, nh_ref, wn_ref, pr_ref):
    np_blk = lax.dot_general(nf_ref[...], wn_ref[...],
                             (((1,), (1,)), ((), ())),
                             preferred_element_type=jnp.float32)
    p = jnp.exp(np_blk - jnp.max(np_blk, axis=1, keepdims=True))
    pr_ref[:, :H] = p
    pr_ref[:, H:] = nh_ref[...] * p


def _edge_prep_body(ef_ref, we_ref, q_ref):
    ep = lax.dot_general(ef_ref[...], we_ref[...],
                         (((1,), (1,)), ((), ())),
                         preferred_element_type=jnp.float32)
    q_ref[...] = jnp.exp(ep - jnp.max(ep, axis=1, keepdims=True))


def _gru_body(hp_ref, nh_ref, wih_ref, whh_ref, bih_ref, bhh_ref, out_ref):
    h_new = hp_ref[0] + hp_ref[1]
    h = nh_ref[...]
    gi = lax.dot_general(h_new, wih_ref[...], (((1,), (1,)), ((), ())),
                         preferred_element_type=jnp.float32) + bih_ref[...]
    gh = lax.dot_general(h, whh_ref[...], (((1,), (1,)), ((), ())),
                         preferred_element_type=jnp.float32) + bhh_ref[...]
    r = jax.nn.sigmoid(gi[:, :H] + gh[:, :H])
    z = jax.nn.sigmoid(gi[:, H:2 * H] + gh[:, H:2 * H])
    n = jnp.tanh(gi[:, 2 * H:] + r * gh[:, 2 * H:])
    out_ref[...] = (1.0 - z) * n + z * h


def _sc_body(pr_hbm, q_hbm, ei_hbm, zero_hbm, out_hbm,
             src_a, dst_a, src_b, dst_b, pr_a, pr_b, q_a, q_b,
             acc_sh, sem_a, sem_b):
    cid = lax.axis_index("c")
    sid = lax.axis_index("s")
    wid = sid * NC + cid

    @pl.when(sid == 0)
    def _():
        pltpu.sync_copy(zero_hbm, acc_sh)

    plsc.subcore_barrier()

    n_chunks = BASE_CHUNKS + jnp.where(wid < EXTRA, 1, 0)

    def chunk_base(k):
        return (wid + k * NW) * CH

    def load_idx(k, src_v, dst_v):
        base = chunk_base(k)
        pltpu.sync_copy(ei_hbm.at[pl.ds(base, CH)], src_v)
        pltpu.sync_copy(ei_hbm.at[pl.ds(E + base, CH)], dst_v)

    def issue_gathers(k, src_v, pr_v, q_v, sem):
        pltpu.async_copy(pr_hbm.at[src_v], pr_v, sem)
        pltpu.async_copy(q_hbm.at[pl.ds(chunk_base(k), CH)], q_v, sem)

    def wait_gathers(src_v, pr_v, q_v, sem):
        pltpu.make_async_copy(pr_hbm.at[src_v], pr_v, sem).wait()
        pltpu.make_async_copy(q_hbm.at[pl.ds(0, CH)], q_v, sem).wait()

    lanes = lax.iota(jnp.int32, L)
    dnums = lax.GatherDimensionNumbers(
        offset_dims=(), collapsed_slice_dims=(0,), start_index_map=(0,))

    def compute_scatter(dst_v, pr_v, q_v):
        @plsc.parallel_loop(0, CH, unroll=1)
        def edge_body(i):
            acc = pr_v[i, pl.ds(0, L)] * q_v[i, pl.ds(0, L)]
            for j in range(1, H // L):
                acc = acc + pr_v[i, pl.ds(j * L, L)] * q_v[i, pl.ds(j * L, L)]
            for step in (1, 2, 4, 8):
                perm = lax.gather(
                    acc, (lanes ^ step).reshape(L, 1), dnums,
                    slice_sizes=(1,),
                    mode=lax.GatherScatterMode.PROMISE_IN_BOUNDS)
                acc = acc + perm
            s = 1.0 / acc
            for j in range(H // L):
                q_v[i, pl.ds(j * L, L)] = (
                    pr_v[i, pl.ds(H + j * L, L)] * q_v[i, pl.ds(j * L, L)] * s)

        pltpu.sync_copy(q_v, acc_sh.at[dst_v], add=True)

    load_idx(0, src_a, dst_a)
    issue_gathers(0, src_a, pr_a, q_a, sem_a)

    def pair_body(kk, carry):
        k0 = 2 * kk
        load_idx(k0 + 1, src_b, dst_b)
        issue_gathers(k0 + 1, src_b, pr_b, q_b, sem_b)
        wait_gathers(src_a, pr_a, q_a, sem_a)
        compute_scatter(dst_a, pr_a, q_a)

        @pl.when(k0 + 2 < n_chunks)
        def _():
            load_idx(k0 + 2, src_a, dst_a)
            issue_gathers(k0 + 2, src_a, pr_a, q_a, sem_a)

        wait_gathers(src_b, pr_b, q_b, sem_b)
        compute_scatter(dst_b, pr_b, q_b)
        return carry

    lax.fori_loop(0, BASE_CHUNKS // 2, pair_body, 0)

    @pl.when(n_chunks > BASE_CHUNKS)
    def _():
        wait_gathers(src_a, pr_a, q_a, sem_a)
        compute_scatter(dst_a, pr_a, q_a)

    plsc.subcore_barrier()
    row0 = sid * ROWS_PER_SUB
    pltpu.sync_copy(acc_sh.at[pl.ds(row0, ROWS_PER_SUB)],
                    out_hbm.at[cid, pl.ds(row0, ROWS_PER_SUB)])

    tail0 = NS * ROWS_PER_SUB
    tail = N - tail0

    @pl.when(sid == 0)
    def _():
        pltpu.sync_copy(acc_sh.at[pl.ds(tail0, tail)],
                        out_hbm.at[cid, pl.ds(tail0, tail)])


_sc_scatter = functools.partial(
    pl.kernel,
    mesh=plsc.VectorSubcoreMesh(core_axis_name="c", subcore_axis_name="s"),
    out_type=jax.ShapeDtypeStruct((NC, N, H), jnp.float32),
    scratch_types=[
        pltpu.VMEM((CH,), jnp.int32),
        pltpu.VMEM((CH,), jnp.int32),
        pltpu.VMEM((CH,), jnp.int32),
        pltpu.VMEM((CH,), jnp.int32),
        pltpu.VMEM((CH, 2 * H), jnp.float32),
        pltpu.VMEM((CH, 2 * H), jnp.float32),
        pltpu.VMEM((CH, H), jnp.float32),
        pltpu.VMEM((CH, H), jnp.float32),
        pltpu.VMEM_SHARED((N, H), jnp.float32),
        pltpu.SemaphoreType.DMA,
        pltpu.SemaphoreType.DMA,
    ],
)(_sc_body)


def kernel(node_feats, edge_feats, node_hidden, edge_index, W_node, W_edge,
           W_ih, W_hh, b_ih, b_hh):
    BN = 1000
    pr_arr = pl.pallas_call(
        _node_prep_body,
        grid=(N // BN,),
        in_specs=[pl.BlockSpec((BN, D), lambda i: (i, 0)),
                  pl.BlockSpec((BN, H), lambda i: (i, 0)),
                  pl.BlockSpec((H, D), lambda i: (0, 0))],
        out_specs=pl.BlockSpec((BN, 2 * H), lambda i: (i, 0)),
        out_shape=jax.ShapeDtypeStruct((N, 2 * H), jnp.float32),
    )(node_feats, node_hidden, W_node)

    BE = 4000
    q_arr = pl.pallas_call(
        _edge_prep_body,
        grid=(E // BE,),
        in_specs=[pl.BlockSpec((BE, DE), lambda i: (i, 0)),
                  pl.BlockSpec((H, DE), lambda i: (0, 0))],
        out_specs=pl.BlockSpec((BE, H), lambda i: (i, 0)),
        out_shape=jax.ShapeDtypeStruct((E, H), jnp.float32),
    )(edge_feats, W_edge)

    zeros = jnp.zeros((N, H), jnp.float32)
    hp = _sc_scatter(pr_arr, q_arr, edge_index.reshape(2 * E), zeros)

    BG = 1000
    out = pl.pallas_call(
        _gru_body,
        grid=(N // BG,),
        in_specs=[pl.BlockSpec((2, BG, H), lambda i: (0, i, 0)),
                  pl.BlockSpec((BG, H), lambda i: (i, 0)),
                  pl.BlockSpec((3 * H, H), lambda i: (0, 0)),
                  pl.BlockSpec((3 * H, H), lambda i: (0, 0)),
                  pl.BlockSpec((1, 3 * H), lambda i: (0, 0)),
                  pl.BlockSpec((1, 3 * H), lambda i: (0, 0))],
        out_specs=pl.BlockSpec((BG, H), lambda i: (i, 0)),
        out_shape=jax.ShapeDtypeStruct((N, H), jnp.float32),
    )(hp, node_hidden, W_ih, W_hh, b_ih.reshape(1, 3 * H),
      b_hh.reshape(1, 3 * H))
    return out


# speedup vs baseline: 4.5977x; 1.0098x over previous
import functools
import jax
import jax.numpy as jnp
from jax import lax
from jax.experimental import pallas as pl
from jax.experimental.pallas import tpu as pltpu
from jax.experimental.pallas import tpu_sc as plsc

N, E, D, DE, H = 10000, 320000, 128, 16, 128
NC, NS, L = 2, 16, 16
NW = NC * NS
CH = 64
NCHUNKS = E // CH
BASE_CHUNKS = NCHUNKS // NW
EXTRA = NCHUNKS % NW
ROWS_PER_SUB = 624


def _node_prep_body(nf_ref, nh_ref, wn_ref, pr_ref):
    np_blk = lax.dot_general(nf_ref[...], wn_ref[...],
                             (((1,), (1,)), ((), ())),
                             preferred_element_type=jnp.float32)
    p = jnp.exp(np_blk - jnp.max(np_blk, axis=1, keepdims=True))
    pr_ref[:, :H] = p
    pr_ref[:, H:] = nh_ref[...] * p


def _edge_prep_body(ef_ref, we_ref, q_ref):
    ep = lax.dot_general(ef_ref[...], we_ref[...],
                         (((1,), (1,)), ((), ())),
                         preferred_element_type=jnp.float32)
    q_ref[...] = jnp.exp(ep - jnp.max(ep, axis=1, keepdims=True))


def _gru_body(hp_ref, nh_ref, wih_ref, whh_ref, bih_ref, bhh_ref, out_ref):
    h_new = hp_ref[0] + hp_ref[1]
    h = nh_ref[...]
    gi = lax.dot_general(h_new, wih_ref[...], (((1,), (1,)), ((), ())),
                         preferred_element_type=jnp.float32) + bih_ref[...]
    gh = lax.dot_general(h, whh_ref[...], (((1,), (1,)), ((), ())),
                         preferred_element_type=jnp.float32) + bhh_ref[...]
    r = jax.nn.sigmoid(gi[:, :H] + gh[:, :H])
    z = jax.nn.sigmoid(gi[:, H:2 * H] + gh[:, H:2 * H])
    n = jnp.tanh(gi[:, 2 * H:] + r * gh[:, 2 * H:])
    out_ref[...] = (1.0 - z) * n + z * h


def _sc_body(pr_hbm, q_hbm, ei_hbm, zero_hbm, out_hbm,
             src_a, dst_a, src_b, dst_b, pr_a, pr_b, q_a, q_b,
             acc_sh, sem_a, sem_b):
    cid = lax.axis_index("c")
    sid = lax.axis_index("s")
    wid = sid * NC + cid

    @pl.when(sid == 0)
    def _():
        pltpu.sync_copy(zero_hbm, acc_sh)

    plsc.subcore_barrier()

    n_chunks = BASE_CHUNKS + jnp.where(wid < EXTRA, 1, 0)

    def chunk_base(k):
        return (wid + k * NW) * CH

    def load_idx(k, src_v, dst_v):
        base = chunk_base(k)
        pltpu.sync_copy(ei_hbm.at[pl.ds(base, CH)], src_v)
        pltpu.sync_copy(ei_hbm.at[pl.ds(E + base, CH)], dst_v)

    def issue_gathers(k, src_v, pr_v, q_v, sem):
        pltpu.async_copy(pr_hbm.at[src_v], pr_v, sem)
        pltpu.async_copy(q_hbm.at[pl.ds(chunk_base(k), CH)], q_v, sem)

    def wait_gathers(src_v, pr_v, q_v, sem):
        pltpu.make_async_copy(pr_hbm.at[src_v], pr_v, sem).wait()
        pltpu.make_async_copy(q_hbm.at[pl.ds(0, CH)], q_v, sem).wait()

    lanes = lax.iota(jnp.int32, L)
    dnums = lax.GatherDimensionNumbers(
        offset_dims=(), collapsed_slice_dims=(0,), start_index_map=(0,))

    def compute_scatter(dst_v, pr_v, q_v):
        @plsc.parallel_loop(0, CH, unroll=1)
        def edge_body(i):
            qs = [q_v[i, pl.ds(j * L, L)] for j in range(H // L)]
            rqs = [pr_v[i, pl.ds(H + j * L, L)] * qs[j] for j in range(H // L)]
            acc = pr_v[i, pl.ds(0, L)] * qs[0]
            for j in range(1, H // L):
                acc = acc + pr_v[i, pl.ds(j * L, L)] * qs[j]
            for step in (1, 2, 4, 8):
                perm = lax.gather(
                    acc, (lanes ^ step).reshape(L, 1), dnums,
                    slice_sizes=(1,),
                    mode=lax.GatherScatterMode.PROMISE_IN_BOUNDS)
                acc = acc + perm
            s = 1.0 / acc
            for j in range(H // L):
                q_v[i, pl.ds(j * L, L)] = rqs[j] * s

        pltpu.sync_copy(q_v, acc_sh.at[dst_v], add=True)

    load_idx(0, src_a, dst_a)
    issue_gathers(0, src_a, pr_a, q_a, sem_a)

    def pair_body(kk, carry):
        k0 = 2 * kk
        load_idx(k0 + 1, src_b, dst_b)
        issue_gathers(k0 + 1, src_b, pr_b, q_b, sem_b)
        wait_gathers(src_a, pr_a, q_a, sem_a)
        compute_scatter(dst_a, pr_a, q_a)

        @pl.when(k0 + 2 < n_chunks)
        def _():
            load_idx(k0 + 2, src_a, dst_a)
            issue_gathers(k0 + 2, src_a, pr_a, q_a, sem_a)

        wait_gathers(src_b, pr_b, q_b, sem_b)
        compute_scatter(dst_b, pr_b, q_b)
        return carry

    lax.fori_loop(0, BASE_CHUNKS // 2, pair_body, 0)

    @pl.when(n_chunks > BASE_CHUNKS)
    def _():
        wait_gathers(src_a, pr_a, q_a, sem_a)
        compute_scatter(dst_a, pr_a, q_a)

    plsc.subcore_barrier()
    row0 = sid * ROWS_PER_SUB
    pltpu.sync_copy(acc_sh.at[pl.ds(row0, ROWS_PER_SUB)],
                    out_hbm.at[cid, pl.ds(row0, ROWS_PER_SUB)])

    tail0 = NS * ROWS_PER_SUB
    tail = N - tail0

    @pl.when(sid == 0)
    def _():
        pltpu.sync_copy(acc_sh.at[pl.ds(tail0, tail)],
                        out_hbm.at[cid, pl.ds(tail0, tail)])


_sc_scatter = functools.partial(
    pl.kernel,
    mesh=plsc.VectorSubcoreMesh(core_axis_name="c", subcore_axis_name="s"),
    out_type=jax.ShapeDtypeStruct((NC, N, H), jnp.float32),
    scratch_types=[
        pltpu.VMEM((CH,), jnp.int32),
        pltpu.VMEM((CH,), jnp.int32),
        pltpu.VMEM((CH,), jnp.int32),
        pltpu.VMEM((CH,), jnp.int32),
        pltpu.VMEM((CH, 2 * H), jnp.float32),
        pltpu.VMEM((CH, 2 * H), jnp.float32),
        pltpu.VMEM((CH, H), jnp.float32),
        pltpu.VMEM((CH, H), jnp.float32),
        pltpu.VMEM_SHARED((N, H), jnp.float32),
        pltpu.SemaphoreType.DMA,
        pltpu.SemaphoreType.DMA,
    ],
)(_sc_body)


def kernel(node_feats, edge_feats, node_hidden, edge_index, W_node, W_edge,
           W_ih, W_hh, b_ih, b_hh):
    BN = 1000
    pr_arr = pl.pallas_call(
        _node_prep_body,
        grid=(N // BN,),
        in_specs=[pl.BlockSpec((BN, D), lambda i: (i, 0)),
                  pl.BlockSpec((BN, H), lambda i: (i, 0)),
                  pl.BlockSpec((H, D), lambda i: (0, 0))],
        out_specs=pl.BlockSpec((BN, 2 * H), lambda i: (i, 0)),
        out_shape=jax.ShapeDtypeStruct((N, 2 * H), jnp.float32),
    )(node_feats, node_hidden, W_node)

    BE = 4000
    q_arr = pl.pallas_call(
        _edge_prep_body,
        grid=(E // BE,),
        in_specs=[pl.BlockSpec((BE, DE), lambda i: (i, 0)),
                  pl.BlockSpec((H, DE), lambda i: (0, 0))],
        out_specs=pl.BlockSpec((BE, H), lambda i: (i, 0)),
        out_shape=jax.ShapeDtypeStruct((E, H), jnp.float32),
    )(edge_feats, W_edge)

    zeros = jnp.zeros((N, H), jnp.float32)
    hp = _sc_scatter(pr_arr, q_arr, edge_index.reshape(2 * E), zeros)

    BG = 1000
    out = pl.pallas_call(
        _gru_body,
        grid=(N // BG,),
        in_specs=[pl.BlockSpec((2, BG, H), lambda i: (0, i, 0)),
                  pl.BlockSpec((BG, H), lambda i: (i, 0)),
                  pl.BlockSpec((3 * H, H), lambda i: (0, 0)),
                  pl.BlockSpec((3 * H, H), lambda i: (0, 0)),
                  pl.BlockSpec((1, 3 * H), lambda i: (0, 0)),
                  pl.BlockSpec((1, 3 * H), lambda i: (0, 0))],
        out_specs=pl.BlockSpec((BG, H), lambda i: (i, 0)),
        out_shape=jax.ShapeDtypeStruct((N, H), jnp.float32),
    )(hp, node_hidden, W_ih, W_hh, b_ih.reshape(1, 3 * H),
      b_hh.reshape(1, 3 * H))
    return out
